# baseline (speedup 1.0000x reference)
import functools

import jax
import jax.numpy as jnp
from jax import lax
from jax.experimental import pallas as pl
from jax.experimental.pallas import tpu as pltpu
from jax.experimental.pallas import tpu_sc as plsc

_LANES = 16
_NC = 2
_NS = 16
_MAXK = 16
_GROUP = 16
_SPAN = _GROUP * _MAXK
_CH = 48
_SPAN_BUF = 6 * _CH


def _seg_ranges_body(action_ref, ends_ref, starts_ref):
    sn = action_ref[...] + 1
    bp = sn.shape[1]
    ends = sn
    k = 1
    while k < bp:
        shifted = jnp.concatenate(
            [jnp.zeros((1, k), jnp.int32), ends[:, :-k]], axis=1)
        ends = ends + shifted
        k *= 2
    ends_ref[...] = ends
    starts_ref[...] = ends - sn


def _make_sc_aggregate(n_rows, d_in, bp):
    segs = bp // _NS
    ngroups = segs // _GROUP
    dh = d_in // _NC

    nchunk = dh // _LANES

    def body(starts_hbm, ends_hbm, nf_hbm, out_hbm, starts_v, ends_v,
             rows_a, rows_b, acc_a, acc_b, isem_a, isem_b, osem_a, osem_b):
        c = lax.axis_index("c")
        s = lax.axis_index("s")
        seg0 = pl.multiple_of(s * segs, segs)
        d0 = pl.multiple_of(c * dh, dh)
        pltpu.sync_copy(starts_hbm.at[pl.ds(seg0, segs)], starts_v)
        pltpu.sync_copy(ends_hbm.at[pl.ds(seg0, segs)], ends_v)

        def group_base(g):
            s16 = starts_v[pl.ds(g * _GROUP, _GROUP)]
            base = jnp.minimum((s16[0] // 8) * 8, n_rows - _SPAN_BUF)
            return pl.multiple_of(base, 8)

        def group_nch(g):
            e16 = ends_v[pl.ds(g * _GROUP, _GROUP)]
            span = e16[_GROUP - 1] - group_base(g)
            return (span + _CH - 1) // _CH

        def issue(g, buf, sem):
            base = group_base(g)

            def chunk(i, carry):
                src_row = pl.multiple_of(base + i * _CH, 8)
                dst_row = pl.multiple_of(i * _CH, 8)
                pltpu.async_copy(
                    nf_hbm.at[pl.ds(src_row, _CH), pl.ds(d0, dh)],
                    buf.at[pl.ds(dst_row, _CH), :], sem)
                return carry

            lax.fori_loop(0, group_nch(g), chunk, 0)

        def wait_rows(g, buf, sem):
            def chunk(i, carry):
                pltpu.make_async_copy(
                    nf_hbm.at[pl.ds(0, _CH), pl.ds(0, dh)],
                    buf.at[pl.ds(0, _CH), :], sem).wait()
                return carry

            lax.fori_loop(0, group_nch(g), chunk, 0)

        def wait_out(acc, sem):
            pltpu.make_async_copy(
                acc, out_hbm.at[pl.ds(0, _GROUP), pl.ds(d0, dh)], sem
            ).wait()

        def compute(g, rows_v, acc_v, osem):
            gbase = g * _GROUP
            s16 = starts_v[pl.ds(gbase, _GROUP)]
            e16 = ends_v[pl.ds(gbase, _GROUP)]
            cnt = e16 - s16
            rel = s16 - group_base(g)

            for j in range(_GROUP):
                r0 = rel[j]
                cj = cnt[j]
                accs = [jnp.zeros((_LANES,), jnp.float32)
                        for _ in range(nchunk)]
                for r in range(_MAXK):
                    pred = r < cj
                    for k in range(nchunk):
                        v = rows_v[r0 + r, pl.ds(k * _LANES, _LANES)]
                        accs[k] = accs[k] + jnp.where(pred, v, 0.0)
                for k in range(nchunk):
                    acc_v[j, pl.ds(k * _LANES, _LANES)] = accs[k]

            orow = pl.multiple_of(seg0 + gbase, _GROUP)
            pltpu.async_copy(
                acc_v, out_hbm.at[pl.ds(orow, _GROUP), pl.ds(d0, dh)], osem)

        issue(0, rows_a, isem_a)

        def pair_body(p, carry):
            g0 = 2 * p
            issue(g0 + 1, rows_b, isem_b)
            wait_rows(g0, rows_a, isem_a)

            @pl.when(p > 0)
            def _():
                wait_out(acc_a, osem_a)

            compute(g0, rows_a, acc_a, osem_a)

            @pl.when(g0 + 2 < ngroups)
            def _():
                issue(g0 + 2, rows_a, isem_a)

            wait_rows(g0 + 1, rows_b, isem_b)

            @pl.when(p > 0)
            def _():
                wait_out(acc_b, osem_b)

            compute(g0 + 1, rows_b, acc_b, osem_b)
            return carry

        lax.fori_loop(0, ngroups // 2, pair_body, 0)
        wait_out(acc_a, osem_a)
        wait_out(acc_b, osem_b)

    mesh = plsc.VectorSubcoreMesh(
        core_axis_name="c", subcore_axis_name="s", num_cores=_NC,
        num_subcores=_NS)
    return functools.partial(
        pl.kernel,
        out_type=jax.ShapeDtypeStruct((bp, d_in), jnp.float32),
        mesh=mesh,
        compiler_params=pltpu.CompilerParams(needs_layout_passes=False),
        scratch_types=[
            pltpu.VMEM((segs,), jnp.int32),
            pltpu.VMEM((segs,), jnp.int32),
            pltpu.VMEM((_SPAN_BUF, dh), jnp.float32),
            pltpu.VMEM((_SPAN_BUF, dh), jnp.float32),
            pltpu.VMEM((_GROUP, dh), jnp.float32),
            pltpu.VMEM((_GROUP, dh), jnp.float32),
            pltpu.SemaphoreType.DMA,
            pltpu.SemaphoreType.DMA,
            pltpu.SemaphoreType.DMA,
            pltpu.SemaphoreType.DMA,
        ],
    )(body)


def _mm_body(x_ref, cnt_ref, w_ref, b_ref, o_ref):
    means = x_ref[...] / cnt_ref[...]
    o_ref[...] = (
        jnp.dot(means, w_ref[...], preferred_element_type=jnp.float32)
        + b_ref[...])


def kernel(action, neighbor_feature, W, b):
    bsz = action.shape[0]
    n_rows, d_in = neighbor_feature.shape
    d_out = W.shape[1]

    block = _NS * _GROUP
    bp = ((bsz + block - 1) // block) * block

    ap = jnp.pad(action.astype(jnp.int32), (0, bp - bsz), constant_values=-1)
    ends, starts = pl.pallas_call(
        _seg_ranges_body,
        out_shape=(
            jax.ShapeDtypeStruct((1, bp), jnp.int32),
            jax.ShapeDtypeStruct((1, bp), jnp.int32),
        ),
    )(ap.reshape(1, bp))

    sums = _make_sc_aggregate(n_rows, d_in, bp)(
        starts.reshape(bp), ends.reshape(bp), neighbor_feature)

    cnt_col = (ap[:bsz] + 1).astype(jnp.float32).reshape(bsz, 1)

    bm = next(x for x in (1000, 500, 200, 100, 50, 1) if bsz % x == 0)
    out = pl.pallas_call(
        _mm_body,
        grid=(bsz // bm,),
        in_specs=[
            pl.BlockSpec((bm, d_in), lambda i: (i, 0)),
            pl.BlockSpec((bm, 1), lambda i: (i, 0)),
            pl.BlockSpec((d_in, d_out), lambda i: (0, 0)),
            pl.BlockSpec((1, d_out), lambda i: (0, 0)),
        ],
        out_specs=pl.BlockSpec((bm, d_out), lambda i: (i, 0)),
        out_shape=jax.ShapeDtypeStruct((bsz, d_out), jnp.float32),
    )(sums, cnt_col, W, b.reshape(1, d_out))

    return out


# speedup vs baseline: 1.9686x; 1.9686x over previous
import functools

import jax
import jax.numpy as jnp
from jax import lax
from jax.experimental import pallas as pl
from jax.experimental.pallas import tpu as pltpu
from jax.experimental.pallas import tpu_sc as plsc

_LANES = 16
_NC = 2
_NS = 16
_MAXK = 16
_GROUP = 16
_SPAN = _GROUP * _MAXK
_CH = 48
_SPAN_BUF = 6 * _CH


def _seg_ranges_body(action_ref, ends_ref, starts_ref):
    sn = action_ref[...] + 1
    bp = sn.shape[1]
    ends = sn
    k = 1
    while k < bp:
        shifted = jnp.concatenate(
            [jnp.zeros((1, k), jnp.int32), ends[:, :-k]], axis=1)
        ends = ends + shifted
        k *= 2
    ends_ref[...] = ends
    starts_ref[...] = ends - sn


def _make_sc_aggregate(n_rows, d_in, bp):
    nworkers = _NC * _NS
    segs = bp // nworkers
    gsz = 8
    ngroups = segs // gsz
    span_buf = 3 * _CH
    nchunk = d_in // _LANES

    def body(starts_hbm, ends_hbm, nf_hbm, out_hbm, starts_v, ends_v,
             rows_a, rows_b, acc_a, acc_b, isem_a, isem_b, osem_a, osem_b):
        c = lax.axis_index("c")
        s = lax.axis_index("s")
        wid = s * _NC + c
        seg0 = pl.multiple_of(wid * segs, segs)
        pltpu.sync_copy(starts_hbm.at[pl.ds(seg0, segs)],
                        starts_v.at[pl.ds(0, segs)])
        pltpu.sync_copy(ends_hbm.at[pl.ds(seg0, segs)],
                        ends_v.at[pl.ds(0, segs)])

        def group_vecs(g):
            s16 = starts_v[pl.ds(g * gsz, _LANES)]
            e16 = ends_v[pl.ds(g * gsz, _LANES)]
            return s16, e16

        def group_base(s16):
            base = jnp.minimum((s16[0] // 8) * 8, n_rows - span_buf)
            return pl.multiple_of(base, 8)

        def group_nch(g):
            s16, e16 = group_vecs(g)
            span = e16[gsz - 1] - group_base(s16)
            return (span + _CH - 1) // _CH

        def issue(g, buf, sem):
            s16, _ = group_vecs(g)
            base = group_base(s16)

            def chunk(i, carry):
                src_row = pl.multiple_of(base + i * _CH, 8)
                dst_row = pl.multiple_of(i * _CH, 8)
                pltpu.async_copy(
                    nf_hbm.at[pl.ds(src_row, _CH), :],
                    buf.at[pl.ds(dst_row, _CH), :], sem)
                return carry

            lax.fori_loop(0, group_nch(g), chunk, 0)

        def wait_rows(g, buf, sem):
            def chunk(i, carry):
                pltpu.make_async_copy(
                    nf_hbm.at[pl.ds(0, _CH), :],
                    buf.at[pl.ds(0, _CH), :], sem).wait()
                return carry

            lax.fori_loop(0, group_nch(g), chunk, 0)

        def wait_out(acc, sem):
            pltpu.make_async_copy(
                acc, out_hbm.at[pl.ds(0, gsz), :], sem).wait()

        def compute(g, rows_v, acc_v, osem):
            s16, e16 = group_vecs(g)
            cnt = e16 - s16
            rel = s16 - group_base(s16)

            for j in range(gsz):
                r0 = rel[j]
                cj = cnt[j]

                def row_body(r, accs, r0=r0):
                    row = r0 + r
                    return tuple(
                        accs[k] + rows_v[row, pl.ds(k * _LANES, _LANES)]
                        for k in range(nchunk))

                accs = lax.fori_loop(
                    0, cj, row_body,
                    tuple(jnp.zeros((_LANES,), jnp.float32)
                          for _ in range(nchunk)))
                for k in range(nchunk):
                    acc_v[j, pl.ds(k * _LANES, _LANES)] = accs[k]

            orow = pl.multiple_of(seg0 + g * gsz, gsz)
            pltpu.async_copy(
                acc_v, out_hbm.at[pl.ds(orow, gsz), :], osem)

        issue(0, rows_a, isem_a)

        def pair_body(p, carry):
            g0 = 2 * p
            issue(g0 + 1, rows_b, isem_b)
            wait_rows(g0, rows_a, isem_a)

            @pl.when(p > 0)
            def _():
                wait_out(acc_a, osem_a)

            compute(g0, rows_a, acc_a, osem_a)

            @pl.when(g0 + 2 < ngroups)
            def _():
                issue(g0 + 2, rows_a, isem_a)

            wait_rows(g0 + 1, rows_b, isem_b)

            @pl.when(p > 0)
            def _():
                wait_out(acc_b, osem_b)

            compute(g0 + 1, rows_b, acc_b, osem_b)
            return carry

        lax.fori_loop(0, ngroups // 2, pair_body, 0)
        wait_out(acc_a, osem_a)
        wait_out(acc_b, osem_b)

    mesh = plsc.VectorSubcoreMesh(
        core_axis_name="c", subcore_axis_name="s", num_cores=_NC,
        num_subcores=_NS)
    return functools.partial(
        pl.kernel,
        out_type=jax.ShapeDtypeStruct((bp, d_in), jnp.float32),
        mesh=mesh,
        compiler_params=pltpu.CompilerParams(needs_layout_passes=False),
        scratch_types=[
            pltpu.VMEM((segs + _LANES,), jnp.int32),
            pltpu.VMEM((segs + _LANES,), jnp.int32),
            pltpu.VMEM((span_buf, d_in), jnp.float32),
            pltpu.VMEM((span_buf, d_in), jnp.float32),
            pltpu.VMEM((gsz, d_in), jnp.float32),
            pltpu.VMEM((gsz, d_in), jnp.float32),
            pltpu.SemaphoreType.DMA,
            pltpu.SemaphoreType.DMA,
            pltpu.SemaphoreType.DMA,
            pltpu.SemaphoreType.DMA,
        ],
    )(body)


def _mm_body(x_ref, cnt_ref, w_ref, b_ref, o_ref):
    means = x_ref[...] / cnt_ref[...]
    o_ref[...] = (
        jnp.dot(means, w_ref[...], preferred_element_type=jnp.float32)
        + b_ref[...])


def kernel(action, neighbor_feature, W, b):
    bsz = action.shape[0]
    n_rows, d_in = neighbor_feature.shape
    d_out = W.shape[1]

    block = _NS * _GROUP
    bp = ((bsz + block - 1) // block) * block

    ap = jnp.pad(action.astype(jnp.int32), (0, bp - bsz), constant_values=-1)
    ends, starts = pl.pallas_call(
        _seg_ranges_body,
        out_shape=(
            jax.ShapeDtypeStruct((1, bp), jnp.int32),
            jax.ShapeDtypeStruct((1, bp), jnp.int32),
        ),
    )(ap.reshape(1, bp))

    sums = _make_sc_aggregate(n_rows, d_in, bp)(
        starts.reshape(bp), ends.reshape(bp), neighbor_feature)

    cnt_col = (ap[:bsz] + 1).astype(jnp.float32).reshape(bsz, 1)

    bm = next(x for x in (1000, 500, 200, 100, 50, 1) if bsz % x == 0)
    out = pl.pallas_call(
        _mm_body,
        grid=(bsz // bm,),
        in_specs=[
            pl.BlockSpec((bm, d_in), lambda i: (i, 0)),
            pl.BlockSpec((bm, 1), lambda i: (i, 0)),
            pl.BlockSpec((d_in, d_out), lambda i: (0, 0)),
            pl.BlockSpec((1, d_out), lambda i: (0, 0)),
        ],
        out_specs=pl.BlockSpec((bm, d_out), lambda i: (i, 0)),
        out_shape=jax.ShapeDtypeStruct((bsz, d_out), jnp.float32),
    )(sums, cnt_col, W, b.reshape(1, d_out))

    return out
